# baseline (speedup 1.0000x reference)
import functools

import jax
import jax.numpy as jnp
from jax import lax
from jax.experimental import pallas as pl
from jax.experimental.pallas import tpu as pltpu
from jax.experimental.pallas import tpu_sc as plsc

EMB = 64
SEQ = 200
NUM_CORES = 2
NUM_SUBCORES = 16
NW = NUM_CORES * NUM_SUBCORES
LANES = 16
CH_SEQ = 2
CHUNK = CH_SEQ * SEQ


def _emb_kernel(n_rows):
    rows_per_w = n_rows // NW
    n_chunks = rows_per_w // CHUNK
    mesh = plsc.VectorSubcoreMesh(core_axis_name="c", subcore_axis_name="s")

    @functools.partial(
        pl.kernel,
        out_type=jax.ShapeDtypeStruct((n_rows, EMB), jnp.float32),
        mesh=mesh,
        compiler_params=pltpu.CompilerParams(use_tc_tiling_on_sc=False),
        scratch_types=[
            pltpu.VMEM((CHUNK,), jnp.int32),
            pltpu.VMEM((CHUNK, EMB), jnp.float32),
            pltpu.VMEM((SEQ, EMB), jnp.float32),
            pltpu.SemaphoreType.DMA,
        ],
    )
    def k(x_hbm, tok_hbm, pos_hbm, out_hbm, idx_v, rows_v, pos_v, sem):
        wid = lax.axis_index("s") * NUM_CORES + lax.axis_index("c")
        base = wid * rows_per_w
        pltpu.sync_copy(pos_hbm, pos_v)

        @pl.loop(0, n_chunks)
        def _chunk(ci):
            start = base + ci * CHUNK
            pltpu.sync_copy(x_hbm.at[pl.ds(start, CHUNK)], idx_v)
            pltpu.async_copy(tok_hbm.at[idx_v], rows_v, sem).wait()

            @pl.loop(0, SEQ)
            def _row(s):
                for j in range(EMB // LANES):
                    sl = pl.ds(j * LANES, LANES)
                    pv = pos_v[s, sl]
                    for c2 in range(CH_SEQ):
                        r = c2 * SEQ + s
                        rows_v[r, sl] = rows_v[r, sl] + pv

            pltpu.sync_copy(rows_v, out_hbm.at[pl.ds(start, CHUNK)])

    return k


def kernel(x, token_table, pos_table):
    b, seq = x.shape
    flat = x.reshape(b * seq)
    out = _emb_kernel(b * seq)(flat, token_table, pos_table)
    return out.reshape(b, seq, EMB)


# speedup vs baseline: 1.0859x; 1.0859x over previous
import functools

import jax
import jax.numpy as jnp
from jax import lax
from jax.experimental import pallas as pl
from jax.experimental.pallas import tpu as pltpu
from jax.experimental.pallas import tpu_sc as plsc

EMB = 64
SEQ = 200
NUM_CORES = 2
NUM_SUBCORES = 16
NW = NUM_CORES * NUM_SUBCORES
LANES = 16
CH_SEQ = 2
CHUNK = CH_SEQ * SEQ


def _emb_kernel(n_rows):
    rows_per_w = n_rows // NW
    n_chunks = rows_per_w // CHUNK
    n_pairs = n_chunks // 2
    mesh = plsc.VectorSubcoreMesh(core_axis_name="c", subcore_axis_name="s")

    @functools.partial(
        pl.kernel,
        out_type=jax.ShapeDtypeStruct((n_rows, EMB), jnp.float32),
        mesh=mesh,
        compiler_params=pltpu.CompilerParams(use_tc_tiling_on_sc=False),
        scratch_types=[
            pltpu.VMEM((CHUNK,), jnp.int32),
            pltpu.VMEM((CHUNK,), jnp.int32),
            pltpu.VMEM((CHUNK, EMB), jnp.float32),
            pltpu.VMEM((CHUNK, EMB), jnp.float32),
            pltpu.VMEM((SEQ, EMB), jnp.float32),
            pltpu.SemaphoreType.DMA,
            pltpu.SemaphoreType.DMA,
            pltpu.SemaphoreType.DMA,
            pltpu.SemaphoreType.DMA,
        ],
    )
    def k(x_hbm, tok_hbm, pos_hbm, out_hbm,
          idx0, idx1, rows0, rows1, pos_v, g0, g1, o0, o1):
        wid = lax.axis_index("s") * NUM_CORES + lax.axis_index("c")
        base = wid * rows_per_w
        pltpu.sync_copy(pos_hbm, pos_v)

        def add_pos(rows_v):
            @pl.loop(0, SEQ)
            def _row(s):
                for j in range(EMB // LANES):
                    sl = pl.ds(j * LANES, LANES)
                    pv = pos_v[s, sl]
                    for c2 in range(CH_SEQ):
                        r = c2 * SEQ + s
                        rows_v[r, sl] = rows_v[r, sl] + pv

        pltpu.sync_copy(x_hbm.at[pl.ds(base, CHUNK)], idx0)
        pltpu.async_copy(tok_hbm.at[idx0], rows0, g0)

        @pl.loop(0, n_pairs)
        def _pair(p):
            c_a = base + (2 * p) * CHUNK

            @pl.when(p > 0)
            def _():
                pltpu.make_async_copy(rows1, out_hbm.at[pl.ds(0, CHUNK)], o1).wait()
            pltpu.sync_copy(x_hbm.at[pl.ds(c_a + CHUNK, CHUNK)], idx1)
            pltpu.async_copy(tok_hbm.at[idx1], rows1, g1)
            pltpu.make_async_copy(tok_hbm.at[idx0], rows0, g0).wait()
            add_pos(rows0)
            pltpu.async_copy(rows0, out_hbm.at[pl.ds(c_a, CHUNK)], o0)

            @pl.when(p < n_pairs - 1)
            def _():
                pltpu.make_async_copy(rows0, out_hbm.at[pl.ds(0, CHUNK)], o0).wait()
                pltpu.sync_copy(x_hbm.at[pl.ds(c_a + 2 * CHUNK, CHUNK)], idx0)
                pltpu.async_copy(tok_hbm.at[idx0], rows0, g0)
            pltpu.make_async_copy(tok_hbm.at[idx1], rows1, g1).wait()
            add_pos(rows1)
            pltpu.async_copy(rows1, out_hbm.at[pl.ds(c_a + CHUNK, CHUNK)], o1)

        pltpu.make_async_copy(rows0, out_hbm.at[pl.ds(0, CHUNK)], o0).wait()
        pltpu.make_async_copy(rows1, out_hbm.at[pl.ds(0, CHUNK)], o1).wait()

    return k


def kernel(x, token_table, pos_table):
    b, seq = x.shape
    flat = x.reshape(b * seq)
    out = _emb_kernel(b * seq)(flat, token_table, pos_table)
    return out.reshape(b, seq, EMB)
